# Initial kernel scaffold; baseline (speedup 1.0000x reference)
import jax
import jax.numpy as jnp
from jax.experimental import pallas as pl


def kernel(x, edge_index, W1, b1, W2, b2, W3, b3):
    raise NotImplementedError("write your pallas kernel here")


# speedup vs baseline: 39.8782x; 39.8782x over previous
import functools

import jax
import jax.numpy as jnp
from jax import lax
from jax.experimental import pallas as pl
from jax.experimental.pallas import tpu as pltpu
from jax.experimental.pallas import tpu_sc as plsc

NN = 10000
NP = 10240
EE = 320000
CH = 128
EP = 327680
NROW = EP // CH
NC, NS = 2, 16
RPT = NROW // (NC * NS)
RPS = NP // NS
HP = 80
BR = 1024

_mesh = plsc.VectorSubcoreMesh(core_axis_name="c", subcore_axis_name="s")
_sc_params = pltpu.CompilerParams(use_tc_tiling_on_sc=False)


def _zero_vmem_1d(ref, n):
    def body(i, _):
        ref[pl.ds(i * 16, 16)] = jnp.zeros((16,), jnp.float32)
        return 0
    lax.fori_loop(0, n // 16, body, 0)


@functools.partial(
    pl.kernel,
    out_type=jax.ShapeDtypeStruct((NC, NP), jnp.float32),
    mesh=_mesh,
    compiler_params=_sc_params,
    scratch_types=[
        pltpu.VMEM((RPT, CH), jnp.int32),
        pltpu.VMEM((CH,), jnp.float32),
        pltpu.VMEM((RPS,), jnp.float32),
        pltpu.VMEM_SHARED((NP,), jnp.float32),
    ],
)
def _sc_deg(dst_hbm, out_hbm, dst_v, ones_v, zero_v, acc):
    c = lax.axis_index("c")
    s = lax.axis_index("s")
    t = c * NS + s
    pltpu.sync_copy(dst_hbm.at[pl.ds(t * RPT, RPT)], dst_v)

    def ones_body(i, _):
        ones_v[pl.ds(i * 16, 16)] = jnp.ones((16,), jnp.float32)
        return 0
    lax.fori_loop(0, CH // 16, ones_body, 0)

    _zero_vmem_1d(zero_v, RPS)
    pltpu.sync_copy(zero_v, acc.at[pl.ds(s * RPS, RPS)])
    plsc.subcore_barrier()

    def body(r, _):
        pltpu.sync_copy(ones_v, acc.at[dst_v.at[r]], add=True)
        return 0
    lax.fori_loop(0, RPT, body, 0)

    plsc.subcore_barrier()
    pltpu.sync_copy(acc.at[pl.ds(s * RPS, RPS)],
                    out_hbm.at[c, pl.ds(s * RPS, RPS)])


@functools.partial(
    pl.kernel,
    out_type=jax.ShapeDtypeStruct((NC, NP, HP), jnp.float32),
    mesh=_mesh,
    compiler_params=_sc_params,
    scratch_types=[
        pltpu.VMEM((RPT, CH), jnp.int32),
        pltpu.VMEM((RPT, CH), jnp.int32),
        pltpu.VMEM((CH, HP), jnp.float32),
        pltpu.VMEM((CH, HP), jnp.float32),
        pltpu.VMEM((CH, HP), jnp.float32),
        pltpu.VMEM_SHARED((NP, HP), jnp.float32),
        pltpu.SemaphoreType.DMA,
        pltpu.SemaphoreType.DMA,
    ],
)
def _sc_wide(tab_hbm, src_hbm, dst_hbm, out_hbm,
             src_v, dst_v, buf0, buf1, zbuf, acc, sem0, sem1):
    c = lax.axis_index("c")
    s = lax.axis_index("s")
    t = c * NS + s
    pltpu.sync_copy(src_hbm.at[pl.ds(t * RPT, RPT)], src_v)
    pltpu.sync_copy(dst_hbm.at[pl.ds(t * RPT, RPT)], dst_v)

    def zrow(i, _):
        def zcol(k, _):
            zbuf[i, pl.ds(k * 16, 16)] = jnp.zeros((16,), jnp.float32)
            return 0
        lax.fori_loop(0, HP // 16, zcol, 0)
        return 0
    lax.fori_loop(0, CH, zrow, 0)
    for k in range(RPS // CH):
        pltpu.sync_copy(zbuf, acc.at[pl.ds(s * RPS + k * CH, CH)])
    plsc.subcore_barrier()

    pltpu.async_copy(tab_hbm.at[src_v.at[0]], buf0, sem0)

    def body(j, _):
        r0 = j * 2
        r1 = r0 + 1
        pltpu.async_copy(tab_hbm.at[src_v.at[r1]], buf1, sem1)
        pltpu.make_async_copy(tab_hbm.at[src_v.at[r0]], buf0, sem0).wait()
        pltpu.sync_copy(buf0, acc.at[dst_v.at[r0]], add=True)

        @pl.when(r1 + 1 < RPT)
        def _():
            pltpu.async_copy(tab_hbm.at[src_v.at[r1 + 1]], buf0, sem0)
        pltpu.make_async_copy(tab_hbm.at[src_v.at[r1]], buf1, sem1).wait()
        pltpu.sync_copy(buf1, acc.at[dst_v.at[r1]], add=True)
        return 0
    lax.fori_loop(0, RPT // 2, body, 0)

    plsc.subcore_barrier()
    for k in range(RPS // CH):
        pltpu.sync_copy(acc.at[pl.ds(s * RPS + k * CH, CH)],
                        out_hbm.at[c, pl.ds(s * RPS + k * CH, CH)])


@functools.partial(
    pl.kernel,
    out_type=jax.ShapeDtypeStruct((NC, NP), jnp.float32),
    mesh=_mesh,
    compiler_params=_sc_params,
    scratch_types=[
        pltpu.VMEM((RPT, CH), jnp.int32),
        pltpu.VMEM((RPT, CH), jnp.int32),
        pltpu.VMEM((CH,), jnp.float32),
        pltpu.VMEM((CH,), jnp.float32),
        pltpu.VMEM((RPS,), jnp.float32),
        pltpu.VMEM_SHARED((NP,), jnp.float32),
        pltpu.SemaphoreType.DMA,
        pltpu.SemaphoreType.DMA,
    ],
)
def _sc_scalar(val_hbm, src_hbm, dst_hbm, out_hbm,
               src_v, dst_v, buf0, buf1, zero_v, acc, sem0, sem1):
    c = lax.axis_index("c")
    s = lax.axis_index("s")
    t = c * NS + s
    pltpu.sync_copy(src_hbm.at[pl.ds(t * RPT, RPT)], src_v)
    pltpu.sync_copy(dst_hbm.at[pl.ds(t * RPT, RPT)], dst_v)

    _zero_vmem_1d(zero_v, RPS)
    pltpu.sync_copy(zero_v, acc.at[pl.ds(s * RPS, RPS)])
    plsc.subcore_barrier()

    pltpu.async_copy(val_hbm.at[src_v.at[0]], buf0, sem0)

    def body(j, _):
        r0 = j * 2
        r1 = r0 + 1
        pltpu.async_copy(val_hbm.at[src_v.at[r1]], buf1, sem1)
        pltpu.make_async_copy(val_hbm.at[src_v.at[r0]], buf0, sem0).wait()
        pltpu.sync_copy(buf0, acc.at[dst_v.at[r0]], add=True)

        @pl.when(r1 + 1 < RPT)
        def _():
            pltpu.async_copy(val_hbm.at[src_v.at[r1 + 1]], buf0, sem0)
        pltpu.make_async_copy(val_hbm.at[src_v.at[r1]], buf1, sem1).wait()
        pltpu.sync_copy(buf1, acc.at[dst_v.at[r1]], add=True)
        return 0
    lax.fori_loop(0, RPT // 2, body, 0)

    plsc.subcore_barrier()
    pltpu.sync_copy(acc.at[pl.ds(s * RPS, RPS)],
                    out_hbm.at[c, pl.ds(s * RPS, RPS)])


def _tc_a_body(x_ref, w_ref, degs_ref, t1s_ref, dinv_ref):
    deg = degs_ref[0] + degs_ref[1] + 1.0
    dinv = lax.rsqrt(deg)
    t1 = jnp.dot(x_ref[...], w_ref[...], preferred_element_type=jnp.float32)
    t1s_ref[...] = t1 * dinv
    dinv_ref[...] = dinv


def _tc_a(xp, w1p, degs):
    return pl.pallas_call(
        _tc_a_body,
        grid=(NP // BR,),
        in_specs=[
            pl.BlockSpec((BR, 128), lambda i: (i, 0)),
            pl.BlockSpec((128, HP), lambda i: (0, 0)),
            pl.BlockSpec((NC, BR, 1), lambda i: (0, i, 0)),
        ],
        out_specs=[
            pl.BlockSpec((BR, HP), lambda i: (i, 0)),
            pl.BlockSpec((BR, 1), lambda i: (i, 0)),
        ],
        out_shape=[
            jax.ShapeDtypeStruct((NP, HP), jnp.float32),
            jax.ShapeDtypeStruct((NP, 1), jnp.float32),
        ],
    )(xp, w1p, degs)


def _tc_b_body(parts_ref, t1s_ref, dinv_ref, b1_ref, w2_ref, w3_ref, ut_ref):
    dinv = dinv_ref[...]
    p1 = dinv * (parts_ref[0] + parts_ref[1] + t1s_ref[...])
    h1r = jnp.maximum(p1 + b1_ref[...], 0.0)
    w23 = jnp.dot(w2_ref[...], w3_ref[...], preferred_element_type=jnp.float32)
    u = jnp.dot(h1r, w23, preferred_element_type=jnp.float32)
    ut_ref[...] = dinv * u


def _tc_b(parts, t1s, dinv, b1p, w2p, w3p):
    return pl.pallas_call(
        _tc_b_body,
        grid=(NP // BR,),
        in_specs=[
            pl.BlockSpec((NC, BR, HP), lambda i: (0, i, 0)),
            pl.BlockSpec((BR, HP), lambda i: (i, 0)),
            pl.BlockSpec((BR, 1), lambda i: (i, 0)),
            pl.BlockSpec((1, HP), lambda i: (0, 0)),
            pl.BlockSpec((HP, 96), lambda i: (0, 0)),
            pl.BlockSpec((96, 1), lambda i: (0, 0)),
        ],
        out_specs=pl.BlockSpec((BR, 1), lambda i: (i, 0)),
        out_shape=jax.ShapeDtypeStruct((NP, 1), jnp.float32),
    )(parts, t1s, dinv, b1p, w2p, w3p)


def _tc_c_body(su_ref, ut_ref, dinv_ref, b2_ref, w3_ref, vt_ref):
    dinv = dinv_ref[...]
    su = su_ref[0] + su_ref[1] + ut_ref[...]
    cterm = jnp.sum(b2_ref[...] * w3_ref[...].reshape(1, 96))
    vt_ref[...] = dinv * (dinv * su + cterm)


def _tc_c(su, ut, dinv, b2p, w3p):
    return pl.pallas_call(
        _tc_c_body,
        grid=(NP // BR,),
        in_specs=[
            pl.BlockSpec((NC, BR, 1), lambda i: (0, i, 0)),
            pl.BlockSpec((BR, 1), lambda i: (i, 0)),
            pl.BlockSpec((BR, 1), lambda i: (i, 0)),
            pl.BlockSpec((1, 96), lambda i: (0, 0)),
            pl.BlockSpec((96, 1), lambda i: (0, 0)),
        ],
        out_specs=pl.BlockSpec((BR, 1), lambda i: (i, 0)),
        out_shape=jax.ShapeDtypeStruct((NP, 1), jnp.float32),
    )(su, ut, dinv, b2p, w3p)


def _tc_d_body(sv_ref, vt_ref, dinv_ref, b3_ref, out_ref):
    sv = sv_ref[0] + sv_ref[1] + vt_ref[...]
    out_ref[...] = dinv_ref[...] * sv + b3_ref[...]


def _tc_d(sv, vt, dinv, b3):
    return pl.pallas_call(
        _tc_d_body,
        grid=(NP // BR,),
        in_specs=[
            pl.BlockSpec((NC, BR, 1), lambda i: (0, i, 0)),
            pl.BlockSpec((BR, 1), lambda i: (i, 0)),
            pl.BlockSpec((BR, 1), lambda i: (i, 0)),
            pl.BlockSpec((1, 1), lambda i: (0, 0)),
        ],
        out_specs=pl.BlockSpec((BR, 1), lambda i: (i, 0)),
        out_shape=jax.ShapeDtypeStruct((NP, 1), jnp.float32),
    )(sv, vt, dinv, b3)


def kernel(x, edge_index, W1, b1, W2, b2, W3, b3):
    f32 = jnp.float32
    xp = jnp.zeros((NP, 128), f32).at[:NN].set(x)
    w1p = jnp.zeros((128, HP), f32).at[:, :71].set(W1)
    b1p = jnp.zeros((1, HP), f32).at[0, :71].set(b1)
    w2p = jnp.zeros((HP, 96), f32).at[:71, :82].set(W2)
    w3p = jnp.zeros((96, 1), f32).at[:82].set(W3)
    b2p = jnp.zeros((1, 96), f32).at[0, :82].set(b2)
    b3p = b3.reshape(1, 1).astype(f32)

    npad = EP - EE
    spread = NN + (jnp.arange(npad, dtype=jnp.int32) % (NP - NN))
    src2d = jnp.concatenate([edge_index[0], spread]).reshape(NROW, CH)
    dst2d = jnp.concatenate([edge_index[1], spread]).reshape(NROW, CH)

    degs = _sc_deg(dst2d)
    t1s, dinv = _tc_a(xp, w1p, degs.reshape(NC, NP, 1))
    parts = _sc_wide(t1s, src2d, dst2d)
    ut = _tc_b(parts, t1s, dinv, b1p, w2p, w3p)
    su = _sc_scalar(ut.reshape(NP), src2d, dst2d)
    vt = _tc_c(su.reshape(NC, NP, 1), ut, dinv, b2p, w3p)
    sv = _sc_scalar(vt.reshape(NP), src2d, dst2d)
    out = _tc_d(sv.reshape(NC, NP, 1), vt, dinv, b3p)
    return out[:NN]


# speedup vs baseline: 58.4763x; 1.4664x over previous
import functools

import jax
import jax.numpy as jnp
from jax import lax
from jax.experimental import pallas as pl
from jax.experimental.pallas import tpu as pltpu
from jax.experimental.pallas import tpu_sc as plsc

NN = 10000
NP = 10240
EE = 320000
CH = 128
EP = 327680
NROW = EP // CH
NC, NS = 2, 16
RPT = NROW // (NC * NS)
RPTF = NROW // NS
RPS = NP // NS
HP = 80
BR = 1024
NB = 4

_mesh = plsc.VectorSubcoreMesh(core_axis_name="c", subcore_axis_name="s")
_sc_params = pltpu.CompilerParams(use_tc_tiling_on_sc=False)


def _zero_vmem_1d(ref, n):
    def body(i, _):
        ref[pl.ds(i * 16, 16)] = jnp.zeros((16,), jnp.float32)
        return 0
    lax.fori_loop(0, n // 16, body, 0)


def _propagate(tab, acc, src_v, dst_v, bufs, gsems, ssems, nrows):
    for k in range(NB):
        pltpu.async_copy(tab.at[src_v.at[k]], bufs[k], gsems[k])

    def body(j, _):
        for k in range(NB):
            r = j * NB + k
            pltpu.make_async_copy(tab.at[src_v.at[r]], bufs[k], gsems[k]).wait()
            pltpu.async_copy(bufs[k], acc.at[dst_v.at[r]], ssems[k], add=True)
        for k in range(NB):
            r = j * NB + k

            @pl.when(r + NB < nrows)
            def _():
                pltpu.make_async_copy(
                    bufs[k], acc.at[dst_v.at[r]], ssems[k]).wait()
                pltpu.async_copy(tab.at[src_v.at[r + NB]], bufs[k], gsems[k])
        return 0
    lax.fori_loop(0, nrows // NB, body, 0)
    for k in range(NB):
        r = nrows - NB + k
        pltpu.make_async_copy(bufs[k], acc.at[dst_v.at[r]], ssems[k]).wait()


@functools.partial(
    pl.kernel,
    out_type=jax.ShapeDtypeStruct((NC, NP), jnp.float32),
    mesh=_mesh,
    compiler_params=_sc_params,
    scratch_types=[
        pltpu.VMEM((RPT, CH), jnp.int32),
        pltpu.VMEM((CH,), jnp.float32),
        pltpu.VMEM((RPS,), jnp.float32),
        pltpu.VMEM_SHARED((NP,), jnp.float32),
    ],
)
def _sc_deg(dst_hbm, out_hbm, dst_v, ones_v, zero_v, acc):
    c = lax.axis_index("c")
    s = lax.axis_index("s")
    t = c * NS + s
    pltpu.sync_copy(dst_hbm.at[pl.ds(t * RPT, RPT)], dst_v)

    def ones_body(i, _):
        ones_v[pl.ds(i * 16, 16)] = jnp.ones((16,), jnp.float32)
        return 0
    lax.fori_loop(0, CH // 16, ones_body, 0)

    _zero_vmem_1d(zero_v, RPS)
    pltpu.sync_copy(zero_v, acc.at[pl.ds(s * RPS, RPS)])
    plsc.subcore_barrier()

    def body(r, _):
        pltpu.sync_copy(ones_v, acc.at[dst_v.at[r]], add=True)
        return 0
    lax.fori_loop(0, RPT, body, 0)

    plsc.subcore_barrier()
    pltpu.sync_copy(acc.at[pl.ds(s * RPS, RPS)],
                    out_hbm.at[c, pl.ds(s * RPS, RPS)])


@functools.partial(
    pl.kernel,
    out_type=jax.ShapeDtypeStruct((NC, NP, HP), jnp.float32),
    mesh=_mesh,
    compiler_params=_sc_params,
    scratch_types=[
        pltpu.VMEM((RPT, CH), jnp.int32),
        pltpu.VMEM((RPT, CH), jnp.int32),
        [pltpu.VMEM((CH, HP), jnp.float32)] * NB,
        pltpu.VMEM((CH, HP), jnp.float32),
        pltpu.VMEM_SHARED((NP, HP), jnp.float32),
        [pltpu.SemaphoreType.DMA] * NB,
        [pltpu.SemaphoreType.DMA] * NB,
    ],
)
def _sc_wide(tab_hbm, src_hbm, dst_hbm, out_hbm,
             src_v, dst_v, bufs, zbuf, acc, gsems, ssems):
    c = lax.axis_index("c")
    s = lax.axis_index("s")
    t = c * NS + s
    pltpu.sync_copy(src_hbm.at[pl.ds(t * RPT, RPT)], src_v)
    pltpu.sync_copy(dst_hbm.at[pl.ds(t * RPT, RPT)], dst_v)

    def zrow(i, _):
        def zcol(k, _):
            zbuf[i, pl.ds(k * 16, 16)] = jnp.zeros((16,), jnp.float32)
            return 0
        lax.fori_loop(0, HP // 16, zcol, 0)
        return 0
    lax.fori_loop(0, CH, zrow, 0)
    for k in range(RPS // CH):
        pltpu.sync_copy(zbuf, acc.at[pl.ds(s * RPS + k * CH, CH)])
    plsc.subcore_barrier()

    _propagate(tab_hbm, acc, src_v, dst_v, bufs, gsems, ssems, RPT)

    plsc.subcore_barrier()
    for k in range(RPS // CH):
        pltpu.sync_copy(acc.at[pl.ds(s * RPS + k * CH, CH)],
                        out_hbm.at[c, pl.ds(s * RPS + k * CH, CH)])


@functools.partial(
    pl.kernel,
    out_type=jax.ShapeDtypeStruct((NP,), jnp.float32),
    mesh=_mesh,
    compiler_params=_sc_params,
    scratch_types=[
        pltpu.VMEM((RPTF, CH), jnp.int32),
        pltpu.VMEM((RPTF, CH), jnp.int32),
        [pltpu.VMEM((CH,), jnp.float32)] * NB,
        pltpu.VMEM((RPS,), jnp.float32),
        pltpu.VMEM((RPS,), jnp.float32),
        pltpu.VMEM((RPS,), jnp.float32),
        pltpu.VMEM((RPS,), jnp.float32),
        pltpu.VMEM((RPS,), jnp.float32),
        pltpu.VMEM((16,), jnp.float32),
        pltpu.VMEM((16,), jnp.float32),
        pltpu.VMEM_SHARED((NP,), jnp.float32),
        pltpu.VMEM_SHARED((NP,), jnp.float32),
        [pltpu.SemaphoreType.DMA] * NB,
        [pltpu.SemaphoreType.DMA] * NB,
    ],
)
def _sc_final(ut_hbm, dinv_hbm, cvec_hbm, b3vec_hbm, src_hbm, dst_hbm,
              out_hbm, src_v, dst_v, bufs, zero_v, ut_v, dinv_v, su_v,
              vt_v, c_v, b3_v, tab_s, acc_s, gsems, ssems):
    c = lax.axis_index("c")
    s = lax.axis_index("s")
    sl = pl.ds(s * RPS, RPS)
    pltpu.sync_copy(src_hbm.at[pl.ds(s * RPTF, RPTF)], src_v)
    pltpu.sync_copy(dst_hbm.at[pl.ds(s * RPTF, RPTF)], dst_v)
    pltpu.sync_copy(ut_hbm.at[sl], ut_v)
    pltpu.sync_copy(dinv_hbm.at[sl], dinv_v)
    pltpu.sync_copy(cvec_hbm, c_v)
    pltpu.sync_copy(b3vec_hbm, b3_v)
    _zero_vmem_1d(zero_v, RPS)
    pltpu.sync_copy(zero_v, acc_s.at[sl])
    pltpu.sync_copy(ut_v, tab_s.at[sl])
    plsc.subcore_barrier()

    _propagate(tab_s, acc_s, src_v, dst_v, bufs, gsems, ssems, RPTF)
    plsc.subcore_barrier()

    pltpu.sync_copy(acc_s.at[sl], su_v)
    cb = c_v[...]

    def vt_body(i, _):
        ix = pl.ds(i * 16, 16)
        dv = dinv_v[ix]
        vt_v[ix] = dv * (dv * (su_v[ix] + ut_v[ix]) + cb)
        return 0
    lax.fori_loop(0, RPS // 16, vt_body, 0)
    pltpu.sync_copy(zero_v, acc_s.at[sl])
    pltpu.sync_copy(vt_v, tab_s.at[sl])
    plsc.subcore_barrier()

    _propagate(tab_s, acc_s, src_v, dst_v, bufs, gsems, ssems, RPTF)
    plsc.subcore_barrier()

    pltpu.sync_copy(acc_s.at[sl], su_v)
    bb = b3_v[...]

    def out_body(i, _):
        ix = pl.ds(i * 16, 16)
        ut_v[ix] = dinv_v[ix] * (su_v[ix] + vt_v[ix]) + bb
        return 0
    lax.fori_loop(0, RPS // 16, out_body, 0)

    @pl.when(c == 0)
    def _():
        pltpu.sync_copy(ut_v, out_hbm.at[sl])


def _tc_a_body(x_ref, w_ref, degs_ref, t1s_ref, dinv_ref):
    deg = degs_ref[0] + degs_ref[1] + 1.0
    dinv = lax.rsqrt(deg)
    t1 = jnp.dot(x_ref[...], w_ref[...], preferred_element_type=jnp.float32)
    t1s_ref[...] = t1 * dinv
    dinv_ref[...] = dinv


def _tc_a(xp, w1p, degs):
    return pl.pallas_call(
        _tc_a_body,
        grid=(NP // BR,),
        in_specs=[
            pl.BlockSpec((BR, 128), lambda i: (i, 0)),
            pl.BlockSpec((128, HP), lambda i: (0, 0)),
            pl.BlockSpec((NC, BR, 1), lambda i: (0, i, 0)),
        ],
        out_specs=[
            pl.BlockSpec((BR, HP), lambda i: (i, 0)),
            pl.BlockSpec((BR, 1), lambda i: (i, 0)),
        ],
        out_shape=[
            jax.ShapeDtypeStruct((NP, HP), jnp.float32),
            jax.ShapeDtypeStruct((NP, 1), jnp.float32),
        ],
    )(xp, w1p, degs)


def _tc_b_body(parts_ref, t1s_ref, dinv_ref, b1_ref, w2_ref, w3_ref, b2_ref,
               ut_ref, misc_ref):
    dinv = dinv_ref[...]
    p1 = dinv * (parts_ref[0] + parts_ref[1] + t1s_ref[...])
    h1r = jnp.maximum(p1 + b1_ref[...], 0.0)
    w23 = jnp.dot(w2_ref[...], w3_ref[...], preferred_element_type=jnp.float32)
    u = jnp.dot(h1r, w23, preferred_element_type=jnp.float32)
    ut_ref[...] = dinv * u
    cterm = jnp.sum(b2_ref[...] * w3_ref[...].reshape(1, 96))
    misc_ref[...] = jnp.full((1, 16), cterm, jnp.float32)


def _tc_b(parts, t1s, dinv, b1p, w2p, w3p, b2p):
    return pl.pallas_call(
        _tc_b_body,
        grid=(NP // BR,),
        in_specs=[
            pl.BlockSpec((NC, BR, HP), lambda i: (0, i, 0)),
            pl.BlockSpec((BR, HP), lambda i: (i, 0)),
            pl.BlockSpec((BR, 1), lambda i: (i, 0)),
            pl.BlockSpec((1, HP), lambda i: (0, 0)),
            pl.BlockSpec((HP, 96), lambda i: (0, 0)),
            pl.BlockSpec((96, 1), lambda i: (0, 0)),
            pl.BlockSpec((1, 96), lambda i: (0, 0)),
        ],
        out_specs=[
            pl.BlockSpec((BR, 1), lambda i: (i, 0)),
            pl.BlockSpec((1, 16), lambda i: (0, 0)),
        ],
        out_shape=[
            jax.ShapeDtypeStruct((NP, 1), jnp.float32),
            jax.ShapeDtypeStruct((1, 16), jnp.float32),
        ],
    )(parts, t1s, dinv, b1p, w2p, w3p, b2p)


def kernel(x, edge_index, W1, b1, W2, b2, W3, b3):
    f32 = jnp.float32
    xp = jnp.zeros((NP, 128), f32).at[:NN].set(x)
    w1p = jnp.zeros((128, HP), f32).at[:, :71].set(W1)
    b1p = jnp.zeros((1, HP), f32).at[0, :71].set(b1)
    w2p = jnp.zeros((HP, 96), f32).at[:71, :82].set(W2)
    w3p = jnp.zeros((96, 1), f32).at[:82].set(W3)
    b2p = jnp.zeros((1, 96), f32).at[0, :82].set(b2)
    b3vec = jnp.broadcast_to(b3.astype(f32), (16,))

    npad = EP - EE
    spread = NN + (jnp.arange(npad, dtype=jnp.int32) % (NP - NN))
    src2d = jnp.concatenate([edge_index[0], spread]).reshape(NROW, CH)
    dst2d = jnp.concatenate([edge_index[1], spread]).reshape(NROW, CH)

    degs = _sc_deg(dst2d)
    t1s, dinv = _tc_a(xp, w1p, degs.reshape(NC, NP, 1))
    parts = _sc_wide(t1s, src2d, dst2d)
    ut, misc = _tc_b(parts, t1s, dinv, b1p, w2p, w3p, b2p)
    out = _sc_final(ut.reshape(NP), dinv.reshape(NP), misc.reshape(16),
                    b3vec, src2d, dst2d)
    return out[:NN].reshape(NN, 1)
